# baseline (speedup 1.0000x reference)
import functools

import jax
import jax.numpy as jnp
from jax.experimental import pallas as pl
from jax.experimental.pallas import tpu as pltpu

_C = 256
_K3 = 3 * _C
_NH = 16


def _level_body(x_ref, wk_ref, bc_ref, wh_ref, bh_ref, o_ref, v_ref, *, H, W, Rb):
    bc = bc_ref[0, :].astype(jnp.float32)
    bh = bh_ref[0, :].astype(jnp.float32)
    nchunk = H // Rb
    for r in range(nchunk):
        base = r * Rb
        lo = 1 if r == 0 else 0
        hi = 1 if r == nchunk - 1 else 0
        if lo:
            v_ref[0:W, :] = jnp.zeros((W, _K3), jnp.bfloat16)
        if hi:
            v_ref[(Rb + 1) * W : (Rb + 2) * W, :] = jnp.zeros((W, _K3), jnp.bfloat16)
        nrows = Rb + 2 - lo - hi
        for dx in range(3):
            src = x_ref[0, base - 1 + lo : base - 1 + lo + nrows, dx : dx + W, :]
            v_ref[lo * W : (lo + nrows) * W, dx * _C : (dx + 1) * _C] = src.reshape(
                nrows * W, _C
            )
        acc = jax.lax.dot_general(
            v_ref[0 : Rb * W, :],
            wk_ref[0],
            (((1,), (0,)), ((), ())),
            preferred_element_type=jnp.float32,
        )
        for dy in (1, 2):
            acc = acc + jax.lax.dot_general(
                v_ref[dy * W : dy * W + Rb * W, :],
                wk_ref[dy],
                (((1,), (0,)), ((), ())),
                preferred_element_type=jnp.float32,
            )
        t = jnp.maximum(acc + bc[None, :], 0.0).astype(jnp.bfloat16)
        head = jax.lax.dot_general(
            t,
            wh_ref[...],
            (((1,), (0,)), ((), ())),
            preferred_element_type=jnp.float32,
        )
        out = head + bh[None, :]
        o_ref[0, base : base + Rb, :, :] = out.reshape(Rb, W, _NH)


def _level_call(xp, wk, bc2, wh, bh2, H, W, Rb):
    N = xp.shape[0]
    Wp = W + 2
    body = functools.partial(_level_body, H=H, W=W, Rb=Rb)
    return pl.pallas_call(
        body,
        grid=(N,),
        in_specs=[
            pl.BlockSpec((1, H, Wp, _C), lambda n: (n, 0, 0, 0)),
            pl.BlockSpec((3, _K3, _C), lambda n: (0, 0, 0)),
            pl.BlockSpec((1, _C), lambda n: (0, 0)),
            pl.BlockSpec((_C, _NH), lambda n: (0, 0)),
            pl.BlockSpec((1, _NH), lambda n: (0, 0)),
        ],
        out_specs=pl.BlockSpec((1, H, W, _NH), lambda n: (n, 0, 0, 0)),
        out_shape=jax.ShapeDtypeStruct((N, H, W, _NH), jnp.float32),
        scratch_shapes=[pltpu.VMEM(((Rb + 2) * W, _K3), jnp.bfloat16)],
    )(xp, wk, bc2, wh, bh2)


_RB = {128: 32, 64: 32, 32: 32, 16: 16, 8: 8}


def kernel(x0, x1, x2, x3, x4, W_conv, b_conv, W_cls, b_cls, W_box, b_box):
    feats = [x0, x1, x2, x3, x4]
    wk = jnp.transpose(W_conv, (2, 3, 1, 0)).reshape(3, _K3, _C).astype(jnp.bfloat16)
    whead = jnp.concatenate(
        [W_cls.reshape(3, _C), W_box.reshape(12, _C)], axis=0
    ).T
    whead = jnp.pad(whead, ((0, 0), (0, _NH - 15))).astype(jnp.bfloat16)
    bhead = jnp.pad(jnp.concatenate([b_cls, b_box]), (0, _NH - 15))
    bc2 = b_conv.reshape(1, _C)
    bh2 = bhead.reshape(1, _NH)

    scores, boxes = [], []
    for x in feats:
        N, _, H, W = x.shape
        xp = jnp.transpose(x, (0, 2, 3, 1)).astype(jnp.bfloat16)
        xp = jnp.pad(xp, ((0, 0), (0, 0), (1, 1), (0, 0)))
        out = _level_call(xp, wk, bc2, whead, bh2, H, W, _RB[H])
        scores.append(jnp.transpose(out[..., :3], (0, 3, 1, 2)))
        boxes.append(jnp.transpose(out[..., 3:15], (0, 3, 1, 2)))
    return tuple(scores) + tuple(boxes)
